# Initial kernel scaffold; baseline (speedup 1.0000x reference)
import jax
import jax.numpy as jnp
from jax.experimental import pallas as pl


def kernel(input_ids, embed_tokens):
    raise NotImplementedError("write your pallas kernel here")


# speedup vs baseline: 1.6176x; 1.6176x over previous
import functools

import jax
import jax.numpy as jnp
from jax import lax
from jax.experimental import pallas as pl
from jax.experimental.pallas import tpu as pltpu
from jax.experimental.pallas import tpu_sc as plsc

VOCAB = 100000
HIDDEN = 2048

NC = 2
NS = 16
NW = NC * NS

B = 4 * 4096
B_PER_W = B // NW
CHUNK = 32
NCHUNK = B_PER_W // CHUNK

_mesh = plsc.VectorSubcoreMesh(core_axis_name="c", subcore_axis_name="s")


@functools.partial(
    pl.kernel,
    out_type=jax.ShapeDtypeStruct((B, HIDDEN), jnp.float32),
    mesh=_mesh,
    scratch_types=[
        pltpu.VMEM((NCHUNK, CHUNK), jnp.int32),
        pltpu.VMEM((CHUNK, HIDDEN), jnp.float32),
        pltpu.SemaphoreType.DMA,
    ],
)
def _embed_lookup(ids_hbm, table_hbm, out_hbm, idx_v, rows_v, gsem):
    wid = lax.axis_index("s") * NC + lax.axis_index("c")
    pltpu.sync_copy(ids_hbm.at[wid], idx_v)

    def step(j, _):
        pltpu.async_copy(table_hbm.at[idx_v.at[j]], rows_v, gsem).wait()
        row0 = (wid * NCHUNK + j) * CHUNK
        pltpu.sync_copy(rows_v, out_hbm.at[pl.ds(row0, CHUNK)])
        return 0

    lax.fori_loop(0, NCHUNK, step, 0)


def kernel(input_ids, embed_tokens):
    ids3 = input_ids.reshape(NW, NCHUNK, CHUNK)
    out = _embed_lookup(ids3, embed_tokens)
    return out.reshape(input_ids.shape[0], input_ids.shape[1], HIDDEN)


# speedup vs baseline: 1.7035x; 1.0531x over previous
import functools

import jax
import jax.numpy as jnp
from jax import lax
from jax.experimental import pallas as pl
from jax.experimental.pallas import tpu as pltpu
from jax.experimental.pallas import tpu_sc as plsc

VOCAB = 100000
HIDDEN = 2048

NC = 2
NS = 16
NW = NC * NS

B = 4 * 4096
B_PER_W = B // NW
CHUNK = 16
NCHUNK = B_PER_W // CHUNK

_mesh = plsc.VectorSubcoreMesh(core_axis_name="c", subcore_axis_name="s")


@functools.partial(
    pl.kernel,
    out_type=jax.ShapeDtypeStruct((B, HIDDEN), jnp.float32),
    mesh=_mesh,
    scratch_types=[
        pltpu.VMEM((NCHUNK, CHUNK), jnp.int32),
        pltpu.VMEM((2, CHUNK, HIDDEN), jnp.float32),
        pltpu.SemaphoreType.DMA,
        pltpu.SemaphoreType.DMA,
    ],
)
def _embed_lookup(ids_hbm, table_hbm, out_hbm, idx_v, rows_v, gsem, ssem):
    wid = lax.axis_index("s") * NC + lax.axis_index("c")
    pltpu.sync_copy(ids_hbm.at[wid], idx_v)
    out_base = wid * NCHUNK

    def gather(c, b):
        pltpu.async_copy(table_hbm.at[idx_v.at[c]], rows_v.at[b], gsem)

    def scatter(c, b):
        row0 = (out_base + c) * CHUNK
        pltpu.async_copy(rows_v.at[b], out_hbm.at[pl.ds(row0, CHUNK)], ssem)

    def wait_gather(b):
        pltpu.make_async_copy(
            table_hbm.at[pl.ds(0, CHUNK)], rows_v.at[b], gsem).wait()

    def wait_scatter(b):
        pltpu.make_async_copy(
            rows_v.at[b], out_hbm.at[pl.ds(0, CHUNK)], ssem).wait()

    gather(0, 0)
    wait_gather(0)
    gather(1, 1)
    scatter(0, 0)

    @pl.loop(1, NCHUNK - 1, step=2)
    def _(j):
        for b in (1, 0):
            c = j if b == 1 else j + 1
            wait_gather(b)
            wait_scatter(1 - b)
            gather(c + 1, 1 - b)
            scatter(c, b)

    wait_gather(1)
    wait_scatter(0)
    scatter(NCHUNK - 1, 1)
    wait_scatter(1)


def kernel(input_ids, embed_tokens):
    ids3 = input_ids.reshape(NW, NCHUNK, CHUNK)
    out = _embed_lookup(ids3, embed_tokens)
    return out.reshape(input_ids.shape[0], input_ids.shape[1], HIDDEN)


# speedup vs baseline: 1.7807x; 1.0453x over previous
import functools

import jax
import jax.numpy as jnp
from jax import lax
from jax.experimental import pallas as pl
from jax.experimental.pallas import tpu as pltpu
from jax.experimental.pallas import tpu_sc as plsc

VOCAB = 100000
HIDDEN = 2048

NC = 2
NS = 16
NW = NC * NS

B = 4 * 4096
B_PER_W = B // NW
CHUNK = 16
NCHUNK = B_PER_W // CHUNK

_mesh = plsc.VectorSubcoreMesh(core_axis_name="c", subcore_axis_name="s")


@functools.partial(
    pl.kernel,
    out_type=jax.ShapeDtypeStruct((B, HIDDEN), jnp.float32),
    mesh=_mesh,
    scratch_types=[
        pltpu.VMEM((NCHUNK, CHUNK), jnp.int32),
        pltpu.VMEM((3, CHUNK, HIDDEN), jnp.float32),
        pltpu.SemaphoreType.DMA,
        pltpu.SemaphoreType.DMA,
    ],
)
def _embed_lookup(ids_hbm, table_hbm, out_hbm, idx_v, rows_v, gsem, ssem):
    wid = lax.axis_index("s") * NC + lax.axis_index("c")
    pltpu.sync_copy(ids_hbm.at[wid], idx_v)
    out_base = wid * NCHUNK

    def gather(c, b):
        pltpu.async_copy(table_hbm.at[idx_v.at[c]], rows_v.at[b], gsem)

    def scatter(c, b):
        row0 = (out_base + c) * CHUNK
        pltpu.async_copy(rows_v.at[b], out_hbm.at[pl.ds(row0, CHUNK)], ssem)

    def wait_gather(b):
        pltpu.make_async_copy(
            table_hbm.at[pl.ds(0, CHUNK)], rows_v.at[b], gsem).wait()

    def wait_scatter(b):
        pltpu.make_async_copy(
            rows_v.at[b], out_hbm.at[pl.ds(0, CHUNK)], ssem).wait()

    gather(0, 0)
    gather(1, 1)
    wait_gather(0)
    gather(2, 2)
    scatter(0, 0)

    @pl.loop(1, NCHUNK - 4, step=3)
    def _(j):
        for b in range(3):
            c = j + b
            wait_gather((1 + b) % 3)
            wait_scatter(b % 3)
            gather(c + 2, b % 3)
            scatter(c, (1 + b) % 3)

    wait_gather(1)
    wait_scatter(0)
    gather(NCHUNK - 2, 0)
    scatter(NCHUNK - 4, 1)

    wait_gather(2)
    wait_scatter(1)
    gather(NCHUNK - 1, 1)
    scatter(NCHUNK - 3, 2)

    wait_gather(0)
    wait_scatter(2)
    scatter(NCHUNK - 2, 0)

    wait_gather(1)
    wait_scatter(0)
    scatter(NCHUNK - 1, 1)
    wait_scatter(1)


def kernel(input_ids, embed_tokens):
    ids3 = input_ids.reshape(NW, NCHUNK, CHUNK)
    out = _embed_lookup(ids3, embed_tokens)
    return out.reshape(input_ids.shape[0], input_ids.shape[1], HIDDEN)
